# baseline (speedup 1.0000x reference)
import functools

import jax
import jax.numpy as jnp
from jax import lax
from jax.experimental import pallas as pl
from jax.experimental.pallas import tpu as pltpu
from jax.experimental.pallas import tpu_sc as plsc

B, N, D = 4, 1024, 768
HID = 3072
E = 10
MIX_HID = 1024
LANES = 16


def _gelu(v):
    return jax.nn.gelu(v, approximate=True)


def _router_body(x_ref, c_ref, ada_w01_ref, ada_b_ref, router_w_ref,
                 ss_ref, lg_ref):
    xm = jnp.mean(x_ref[...], axis=1)
    cc = c_ref[...]
    sc = cc * jax.nn.sigmoid(cc)
    ab = ada_b_ref[...]
    shift_mom = jax.lax.dot_general(sc, ada_w01_ref[0], (((1,), (1,)), ((), ())),
                                    preferred_element_type=jnp.float32)
    shift_mom = shift_mom + ab[:, 0:D]
    scale_mom = jax.lax.dot_general(sc, ada_w01_ref[1], (((1,), (1,)), ((), ())),
                                    preferred_element_type=jnp.float32)
    scale_mom = scale_mom + ab[:, D:2 * D]
    ss_ref[:, 0:D] = shift_mom
    ss_ref[:, D:2 * D] = scale_mom
    ri = xm * (1.0 + scale_mom) + shift_mom
    logits = jax.lax.dot_general(ri, router_w_ref[...], (((1,), (1,)), ((), ())),
                                 preferred_element_type=jnp.float32)
    lg_ref[:, 0:E] = logits
    lg_ref[:, E:LANES] = jnp.full((B, LANES - E), -1e30, jnp.float32)


def _ada2_body(c_ref, ada_w23_ref, ada_w45_ref, ada_b_ref, g_ref):
    cc = c_ref[...]
    sc = cc * jax.nn.sigmoid(cc)
    ab = ada_b_ref[...]
    for k, wref in ((0, ada_w23_ref), (1, ada_w23_ref),
                    (2, ada_w45_ref), (3, ada_w45_ref)):
        g = jax.lax.dot_general(sc, wref[k % 2], (((1,), (1,)), ((), ())),
                                preferred_element_type=jnp.float32)
        g_ref[:, k * D:(k + 1) * D] = g + ab[:, (2 + k) * D:(3 + k) * D]


def _sc_router_body(lg_hbm, idx_hbm, aux_hbm, lg_v, idx_v, aux_v):
    info = plsc.get_sparse_core_info()
    wid = lax.axis_index("s") * info.num_cores + lax.axis_index("c")

    @pl.when(wid == 0)
    def _():
        pltpu.sync_copy(lg_hbm, lg_v)
        lane = lax.iota(jnp.int32, LANES)

        def take16(v, i):
            return lax.gather(
                v, i[:, None],
                lax.GatherDimensionNumbers(offset_dims=(),
                                           collapsed_slice_dims=(0,),
                                           start_index_map=(0,)),
                (1,), mode=lax.GatherScatterMode.PROMISE_IN_BOUNDS)

        def bfly(v, op):
            for k in (1, 2, 4, 8):
                v = op(v, take16(v, lane ^ k))
            return v

        psum = jnp.zeros((LANES,), jnp.float32)
        cnt = jnp.zeros((LANES,), jnp.float32)
        idxv = jnp.zeros((LANES,), jnp.int32)
        for b in range(B):
            row = lg_v[b]
            bmax = bfly(row, jnp.maximum)
            p = jnp.exp(row - bmax)
            pr = p / bfly(p, jnp.add)
            psum = psum + pr
            im = jnp.where(row == bmax, lane, LANES)
            top = bfly(im, jnp.minimum)
            onehot = jnp.where(lane == top, 1.0, 0.0)
            cnt = cnt + onehot
            idxv = jnp.where(lane == b, top, idxv)
        aux_vec = psum * cnt * (E / (B * B))
        aux_out = bfly(aux_vec, jnp.add)
        idx_v[...] = idxv
        aux_v[...] = aux_out
        pltpu.sync_copy(idx_v, idx_hbm)
        pltpu.sync_copy(aux_v, aux_hbm)


_sc_router = functools.partial(
    pl.kernel,
    _sc_router_body,
    out_type=(
        jax.ShapeDtypeStruct((LANES,), jnp.int32),
        jax.ShapeDtypeStruct((LANES,), jnp.float32),
    ),
    mesh=plsc.VectorSubcoreMesh(core_axis_name="c", subcore_axis_name="s"),
    scratch_types=[
        pltpu.VMEM((B, LANES), jnp.float32),
        pltpu.VMEM((LANES,), jnp.int32),
        pltpu.VMEM((LANES,), jnp.float32),
    ],
)


def _block_body(idx_ref, x_ref, ss_ref, g_ref, w1_ref, b1_ref, w2_ref, b2_ref,
                out_w_ref, out_b_ref, fc1_ref, fc1_b_ref, fc2_ref, fc2_b_ref,
                out_ref):
    del idx_ref
    xb = x_ref[0]
    a = ss_ref[0]
    shift = a[:, 0:D]
    scale = a[:, D:2 * D]
    g4 = g_ref[0]
    gate_mom = g4[:, 0:D]
    shift_mlp = g4[:, D:2 * D]
    scale_mlp = g4[:, 2 * D:3 * D]
    gate_mlp = g4[:, 3 * D:4 * D]

    mx = xb * (1.0 + scale) + shift
    mu = jnp.mean(mx, axis=0, keepdims=True)
    var = jnp.mean((mx - mu) ** 2, axis=0, keepdims=True)
    xn = (mx - mu) * jax.lax.rsqrt(var + 1e-5)
    w1 = w1_ref[0]
    h = jax.lax.dot_general(xn, w1, (((0,), (1,)), ((), ())),
                            preferred_element_type=jnp.float32)
    g = _gelu(h + b1_ref[0])
    w2 = w2_ref[0]
    ot = jax.lax.dot_general(g, w2, (((1,), (1,)), ((), ())),
                             preferred_element_type=jnp.float32)
    ot = ot + b2_ref[0]

    y = jax.lax.dot_general(ot, out_w_ref[...], (((0,), (1,)), ((), ())),
                            preferred_element_type=jnp.float32)
    y = y + out_b_ref[...]
    TT = N // 2
    for t in range(2):
        x1 = xb[t * TT:(t + 1) * TT, :] + gate_mom * y[t * TT:(t + 1) * TT, :]
        mu2 = jnp.mean(x1, axis=1, keepdims=True)
        var2 = jnp.mean((x1 - mu2) ** 2, axis=1, keepdims=True)
        xn2 = (x1 - mu2) * jax.lax.rsqrt(var2 + 1e-6)
        mod = xn2 * (1.0 + scale_mlp) + shift_mlp
        hm = jax.lax.dot_general(mod, fc1_ref[...], (((1,), (1,)), ((), ())),
                                 preferred_element_type=jnp.float32)
        gm = _gelu(hm + fc1_b_ref[...])
        mlp = jax.lax.dot_general(gm, fc2_ref[...], (((1,), (1,)), ((), ())),
                                  preferred_element_type=jnp.float32)
        mlp = mlp + fc2_b_ref[...]
        out_ref[0, t * TT:(t + 1) * TT, :] = x1 + gate_mlp * mlp


@jax.jit
def kernel(x, c, ada_w, ada_b, router_w, out_w, out_b,
           exp_fc1_w, exp_fc1_b, exp_fc2_w, exp_fc2_b,
           mlp_fc1_w, mlp_fc1_b, mlp_fc2_w, mlp_fc2_b):
    f32 = jnp.float32

    ada_w6 = ada_w.reshape(6, D, D)
    ada_b2 = ada_b.reshape(1, 6 * D)

    ss, logits_pad = pl.pallas_call(
        _router_body,
        in_specs=[
            pl.BlockSpec((B, N, D), lambda i: (0, 0, 0)),
            pl.BlockSpec((B, D), lambda i: (0, 0)),
            pl.BlockSpec((2, D, D), lambda i: (0, 0, 0)),
            pl.BlockSpec((1, 6 * D), lambda i: (0, 0)),
            pl.BlockSpec((E, D), lambda i: (0, 0)),
        ],
        out_specs=(
            pl.BlockSpec((B, 2 * D), lambda i: (0, 0)),
            pl.BlockSpec((B, LANES), lambda i: (0, 0)),
        ),
        out_shape=(
            jax.ShapeDtypeStruct((B, 2 * D), f32),
            jax.ShapeDtypeStruct((B, LANES), f32),
        ),
        grid=(1,),
    )(x, c, ada_w6, ada_b2, router_w)
    ss3 = ss.reshape(B, 1, 2 * D)

    idx16, aux16 = _sc_router()(logits_pad)

    g4 = pl.pallas_call(
        _ada2_body,
        in_specs=[
            pl.BlockSpec((B, D), lambda i: (0, 0)),
            pl.BlockSpec((2, D, D), lambda i: (1, 0, 0)),
            pl.BlockSpec((2, D, D), lambda i: (2, 0, 0)),
            pl.BlockSpec((1, 6 * D), lambda i: (0, 0)),
        ],
        out_specs=pl.BlockSpec((B, 4 * D), lambda i: (0, 0)),
        out_shape=jax.ShapeDtypeStruct((B, 4 * D), f32),
        grid=(1,),
    )(c, ada_w6, ada_w6, ada_b2)
    g43 = g4.reshape(B, 1, 4 * D)

    grid_b = pltpu.PrefetchScalarGridSpec(
        num_scalar_prefetch=1,
        grid=(B,),
        in_specs=[
            pl.BlockSpec((1, N, D), lambda b, idx_ref: (b, 0, 0)),
            pl.BlockSpec((1, 1, 2 * D), lambda b, idx_ref: (b, 0, 0)),
            pl.BlockSpec((1, 1, 4 * D), lambda b, idx_ref: (b, 0, 0)),
            pl.BlockSpec((1, MIX_HID, N), lambda b, idx_ref: (idx_ref[b], 0, 0)),
            pl.BlockSpec((1, 1, MIX_HID), lambda b, idx_ref: (idx_ref[b], 0, 0)),
            pl.BlockSpec((1, N, MIX_HID), lambda b, idx_ref: (idx_ref[b], 0, 0)),
            pl.BlockSpec((1, 1, N), lambda b, idx_ref: (idx_ref[b], 0, 0)),
            pl.BlockSpec((D, D), lambda b, idx_ref: (0, 0)),
            pl.BlockSpec((1, D), lambda b, idx_ref: (0, 0)),
            pl.BlockSpec((HID, D), lambda b, idx_ref: (0, 0)),
            pl.BlockSpec((1, HID), lambda b, idx_ref: (0, 0)),
            pl.BlockSpec((D, HID), lambda b, idx_ref: (0, 0)),
            pl.BlockSpec((1, D), lambda b, idx_ref: (0, 0)),
        ],
        out_specs=pl.BlockSpec((1, N, D), lambda b, idx_ref: (b, 0, 0)),
    )
    x2 = pl.pallas_call(
        _block_body,
        grid_spec=grid_b,
        out_shape=jax.ShapeDtypeStruct((B, N, D), f32),
        compiler_params=pltpu.CompilerParams(vmem_limit_bytes=100 * 1024 * 1024),
    )(idx16, x, ss3, g43, exp_fc1_w, exp_fc1_b.reshape(E, 1, MIX_HID),
      exp_fc2_w, exp_fc2_b.reshape(E, 1, N),
      out_w, out_b.reshape(1, D),
      mlp_fc1_w, mlp_fc1_b.reshape(1, HID), mlp_fc2_w, mlp_fc2_b.reshape(1, D))

    return (x2, aux16[0])


# speedup vs baseline: 1.0395x; 1.0395x over previous
import functools

import jax
import jax.numpy as jnp
from jax import lax
from jax.experimental import pallas as pl
from jax.experimental.pallas import tpu as pltpu
from jax.experimental.pallas import tpu_sc as plsc

B, N, D = 4, 1024, 768
HID = 3072
E = 10
MIX_HID = 1024
LANES = 16


def _gelu(v):
    return jax.nn.gelu(v, approximate=True)


def _router_body(x_ref, c_ref, ada_w_ref, ada_b_ref, router_w_ref,
                 ada_ref, lg_ref):
    xm = jnp.mean(x_ref[...], axis=1)
    cc = c_ref[...]
    sc = cc * jax.nn.sigmoid(cc)
    ada = jax.lax.dot_general(sc, ada_w_ref[...], (((1,), (1,)), ((), ())),
                              preferred_element_type=jnp.float32)
    ada = ada + ada_b_ref[...]
    ada_ref[...] = ada
    shift_mom = ada[:, 0:D]
    scale_mom = ada[:, D:2 * D]
    ri = xm * (1.0 + scale_mom) + shift_mom
    logits = jax.lax.dot_general(ri, router_w_ref[...], (((1,), (1,)), ((), ())),
                                 preferred_element_type=jnp.float32)
    lg_ref[:, 0:E] = logits
    lg_ref[:, E:LANES] = jnp.full((B, LANES - E), -1e30, jnp.float32)


def _sc_router_body(lg_hbm, idx_hbm, aux_hbm, lg_v, idx_v, aux_v):
    info = plsc.get_sparse_core_info()
    wid = lax.axis_index("s") * info.num_cores + lax.axis_index("c")

    @pl.when(wid == 0)
    def _():
        pltpu.sync_copy(lg_hbm, lg_v)
        lane = lax.iota(jnp.int32, LANES)

        def take16(v, i):
            return lax.gather(
                v, i[:, None],
                lax.GatherDimensionNumbers(offset_dims=(),
                                           collapsed_slice_dims=(0,),
                                           start_index_map=(0,)),
                (1,), mode=lax.GatherScatterMode.PROMISE_IN_BOUNDS)

        def bfly(v, op):
            for k in (1, 2, 4, 8):
                v = op(v, take16(v, lane ^ k))
            return v

        psum = jnp.zeros((LANES,), jnp.float32)
        cnt = jnp.zeros((LANES,), jnp.float32)
        idxv = jnp.zeros((LANES,), jnp.int32)
        for b in range(B):
            row = lg_v[b]
            bmax = bfly(row, jnp.maximum)
            p = jnp.exp(row - bmax)
            pr = p / bfly(p, jnp.add)
            psum = psum + pr
            im = jnp.where(row == bmax, lane, LANES)
            top = bfly(im, jnp.minimum)
            onehot = jnp.where(lane == top, 1.0, 0.0)
            cnt = cnt + onehot
            idxv = jnp.where(lane == b, top, idxv)
        aux_vec = psum * cnt * (E / (B * B))
        aux_out = bfly(aux_vec, jnp.add)
        idx_v[...] = idxv
        aux_v[...] = aux_out
        pltpu.sync_copy(idx_v, idx_hbm)
        pltpu.sync_copy(aux_v, aux_hbm)


_sc_router = functools.partial(
    pl.kernel,
    _sc_router_body,
    out_type=(
        jax.ShapeDtypeStruct((LANES,), jnp.int32),
        jax.ShapeDtypeStruct((LANES,), jnp.float32),
    ),
    mesh=plsc.VectorSubcoreMesh(core_axis_name="c", subcore_axis_name="s"),
    scratch_types=[
        pltpu.VMEM((B, LANES), jnp.float32),
        pltpu.VMEM((LANES,), jnp.int32),
        pltpu.VMEM((LANES,), jnp.float32),
    ],
)


def _block_body(idx_ref, x_ref, ada_ref, w1_ref, b1_ref, w2_ref, b2_ref,
                out_w_ref, out_b_ref, fc1_ref, fc1_b_ref, fc2_ref, fc2_b_ref,
                out_ref):
    del idx_ref
    xb = x_ref[0]
    a = ada_ref[0]
    shift = a[:, 0:D]
    scale = a[:, D:2 * D]
    gate_mom = a[:, 2 * D:3 * D]
    shift_mlp = a[:, 3 * D:4 * D]
    scale_mlp = a[:, 4 * D:5 * D]
    gate_mlp = a[:, 5 * D:6 * D]

    mx = xb * (1.0 + scale) + shift
    mu = jnp.mean(mx, axis=0, keepdims=True)
    var = jnp.mean((mx - mu) ** 2, axis=0, keepdims=True)
    xn = (mx - mu) * jax.lax.rsqrt(var + 1e-5)
    w1 = w1_ref[0]
    h = jax.lax.dot_general(xn, w1, (((0,), (1,)), ((), ())),
                            preferred_element_type=jnp.float32)
    g = _gelu(h + b1_ref[0])
    w2 = w2_ref[0]
    ot = jax.lax.dot_general(g, w2, (((1,), (1,)), ((), ())),
                             preferred_element_type=jnp.float32)
    ot = ot + b2_ref[0]

    y = jax.lax.dot_general(ot, out_w_ref[...], (((0,), (1,)), ((), ())),
                            preferred_element_type=jnp.float32)
    y = y + out_b_ref[...]
    TT = N // 2
    for t in range(2):
        x1 = xb[t * TT:(t + 1) * TT, :] + gate_mom * y[t * TT:(t + 1) * TT, :]
        mu2 = jnp.mean(x1, axis=1, keepdims=True)
        var2 = jnp.mean((x1 - mu2) ** 2, axis=1, keepdims=True)
        xn2 = (x1 - mu2) * jax.lax.rsqrt(var2 + 1e-6)
        mod = xn2 * (1.0 + scale_mlp) + shift_mlp
        hm = jax.lax.dot_general(mod, fc1_ref[...], (((1,), (1,)), ((), ())),
                                 preferred_element_type=jnp.float32)
        gm = _gelu(hm + fc1_b_ref[...])
        mlp = jax.lax.dot_general(gm, fc2_ref[...], (((1,), (1,)), ((), ())),
                                  preferred_element_type=jnp.float32)
        mlp = mlp + fc2_b_ref[...]
        out_ref[0, t * TT:(t + 1) * TT, :] = x1 + gate_mlp * mlp


@jax.jit
def kernel(x, c, ada_w, ada_b, router_w, out_w, out_b,
           exp_fc1_w, exp_fc1_b, exp_fc2_w, exp_fc2_b,
           mlp_fc1_w, mlp_fc1_b, mlp_fc2_w, mlp_fc2_b):
    f32 = jnp.float32

    ada, logits_pad = pl.pallas_call(
        _router_body,
        out_shape=(
            jax.ShapeDtypeStruct((B, 6 * D), f32),
            jax.ShapeDtypeStruct((B, LANES), f32),
        ),
    )(x, c, ada_w, ada_b.reshape(1, 6 * D), router_w)
    ada3 = ada.reshape(B, 1, 6 * D)

    idx16, aux16 = _sc_router()(logits_pad)

    grid_b = pltpu.PrefetchScalarGridSpec(
        num_scalar_prefetch=1,
        grid=(B,),
        in_specs=[
            pl.BlockSpec((1, N, D), lambda b, idx_ref: (b, 0, 0)),
            pl.BlockSpec((1, 1, 6 * D), lambda b, idx_ref: (b, 0, 0)),
            pl.BlockSpec((1, MIX_HID, N), lambda b, idx_ref: (idx_ref[b], 0, 0)),
            pl.BlockSpec((1, 1, MIX_HID), lambda b, idx_ref: (idx_ref[b], 0, 0)),
            pl.BlockSpec((1, N, MIX_HID), lambda b, idx_ref: (idx_ref[b], 0, 0)),
            pl.BlockSpec((1, 1, N), lambda b, idx_ref: (idx_ref[b], 0, 0)),
            pl.BlockSpec((D, D), lambda b, idx_ref: (0, 0)),
            pl.BlockSpec((1, D), lambda b, idx_ref: (0, 0)),
            pl.BlockSpec((HID, D), lambda b, idx_ref: (0, 0)),
            pl.BlockSpec((1, HID), lambda b, idx_ref: (0, 0)),
            pl.BlockSpec((D, HID), lambda b, idx_ref: (0, 0)),
            pl.BlockSpec((1, D), lambda b, idx_ref: (0, 0)),
        ],
        out_specs=pl.BlockSpec((1, N, D), lambda b, idx_ref: (b, 0, 0)),
    )
    x2 = pl.pallas_call(
        _block_body,
        grid_spec=grid_b,
        out_shape=jax.ShapeDtypeStruct((B, N, D), f32),
        compiler_params=pltpu.CompilerParams(vmem_limit_bytes=100 * 1024 * 1024),
    )(idx16, x, ada3, exp_fc1_w, exp_fc1_b.reshape(E, 1, MIX_HID),
      exp_fc2_w, exp_fc2_b.reshape(E, 1, N),
      out_w, out_b.reshape(1, D),
      mlp_fc1_w, mlp_fc1_b.reshape(1, HID), mlp_fc2_w, mlp_fc2_b.reshape(1, D))

    return (x2, aux16[0])
